# baseline (speedup 1.0000x reference)
import jax
import jax.numpy as jnp
from jax.experimental import pallas as pl

THRESH = 0.1

_TN = (((0,), (0,)), ((), ()))
_NN = (((1,), (0,)), ((), ()))


def _dot(a, b, dims):
    return jax.lax.dot_general(a, b, dims, preferred_element_type=jnp.float32)


def _normalize_cols(x):
    n = jnp.maximum(jnp.sqrt(jnp.sum(x * x, axis=0, keepdims=True)), 1e-8)
    return x / n


def _hg_kernel(xt_ref, xc1_ref, xc2_ref, w1_ref, b1_ref, w2_ref, b2_ref,
               yt_ref, yc1_ref, yc2_ref):
    w1 = w1_ref[...]
    b1 = b1_ref[...]
    w2 = w2_ref[...]
    b2 = b2_ref[...]
    nc = w1.shape[0]

    selfs = (xt_ref[0], xc2_ref[0])
    nbrs = ((xt_ref[1], xc1_ref[0]), (xc1_ref[1], xc2_ref[1]))

    for i in range(2):
        tn = _normalize_cols(xt_ref[i])
        ca = _normalize_cols(xc1_ref[i])
        cb = _normalize_cols(xc2_ref[i])

        m_a = (_dot(tn, ca, _TN) > THRESH).astype(jnp.float32)
        m_b = (_dot(tn, cb, _TN) > THRESH).astype(jnp.float32)
        mt_a = m_a.T
        mt_b = m_b.T

        t_self = _dot(w1, selfs[i], _TN) + b1
        t_na = _dot(w1, nbrs[i][0], _TN) + b1
        t_nb = _dot(w1, nbrs[i][1], _TN) + b1

        s = t_self + _dot(t_na, mt_a, _NN) + _dot(t_nb, mt_b, _NN)
        x_edge = s[:nc] / s[nc:nc + 1]
        u = _dot(w2, x_edge, _TN) + b2

        stu_a = _dot(u, m_a, _NN)
        stu_b = _dot(u, m_b, _NN)
        s_a = stu_a[:nc] / jnp.maximum(stu_a[nc:nc + 1], 1.0)
        s_b = stu_b[:nc] / jnp.maximum(stu_b[nc:nc + 1], 1.0)
        u = u[:nc]

        if i == 0:
            yt_ref[0] = u
            yt_ref[1] = s_a
            yc1_ref[0] = s_b
        else:
            yc2_ref[0] = u
            yc1_ref[1] = s_a
            yc2_ref[1] = s_b


def kernel(X_target, X_context1, X_context2, W1, b1, W2, b2):
    B, C, Hh, Ww = X_target.shape
    N = Hh * Ww
    xt = X_target.reshape(B, C, N)
    xc1 = X_context1.reshape(B, C, N)
    xc2 = X_context2.reshape(B, C, N)
    zcol = jnp.zeros((C, 1), jnp.float32)
    one = jnp.ones((1, 1), jnp.float32)
    w1e = jnp.concatenate([W1, zcol], axis=1)
    w2e = jnp.concatenate([W2, zcol], axis=1)
    b1e = jnp.concatenate([b1.reshape(C, 1), one], axis=0)
    b2e = jnp.concatenate([b2.reshape(C, 1), one], axis=0)

    shp = jax.ShapeDtypeStruct((B, C, N), jnp.float32)
    yt, yc1, yc2 = pl.pallas_call(
        _hg_kernel,
        out_shape=[shp, shp, shp],
    )(xt, xc1, xc2, w1e, b1e, w2e, b2e)

    rs = lambda a: a.reshape(B, C, Hh, Ww)
    return (rs(yt), rs(yc1), rs(yc2))


# speedup vs baseline: 1.0374x; 1.0374x over previous
import jax
import jax.numpy as jnp
from jax.experimental import pallas as pl

THRESH = 0.1

_TN = (((0,), (0,)), ((), ()))
_NN = (((1,), (0,)), ((), ()))


def _dot(a, b, dims):
    return jax.lax.dot_general(a, b, dims, preferred_element_type=jnp.float32)


def _normalize_cols(x):
    n = jnp.maximum(jnp.sqrt(jnp.sum(x * x, axis=0, keepdims=True)), 1e-8)
    return x / n


def _hg_kernel(xt_ref, xc1_ref, xc2_ref, w1_ref, b1_ref, w2_ref, b2_ref,
               yt_ref, yc1_ref, yc2_ref):
    w1 = w1_ref[...]
    b1 = b1_ref[...]
    w2 = w2_ref[...]
    b2 = b2_ref[...]

    selfs = (xt_ref[0], xc2_ref[0])
    nbrs = ((xt_ref[1], xc1_ref[0]), (xc1_ref[1], xc2_ref[1]))

    for i in range(2):
        tn = _normalize_cols(xt_ref[i])
        ca = _normalize_cols(xc1_ref[i])
        cb = _normalize_cols(xc2_ref[i])

        m_a = (_dot(tn, ca, _TN) > THRESH).astype(jnp.float32)
        m_b = (_dot(tn, cb, _TN) > THRESH).astype(jnp.float32)
        mt_a = m_a.T
        mt_b = m_b.T

        d_e = (1.0 + jnp.sum(mt_a, axis=0, keepdims=True)
               + jnp.sum(mt_b, axis=0, keepdims=True))

        t_self = _dot(w1, selfs[i], _TN) + b1
        t_na = _dot(w1, nbrs[i][0], _TN) + b1
        t_nb = _dot(w1, nbrs[i][1], _TN) + b1

        x_edge = (t_self + _dot(t_na, mt_a, _NN) + _dot(t_nb, mt_b, _NN)) / d_e
        u = _dot(w2, x_edge, _TN) + b2

        d_va = jnp.maximum(jnp.sum(m_a, axis=0, keepdims=True), 1.0)
        d_vb = jnp.maximum(jnp.sum(m_b, axis=0, keepdims=True), 1.0)
        s_a = _dot(u, m_a, _NN) / d_va
        s_b = _dot(u, m_b, _NN) / d_vb

        if i == 0:
            yt_ref[0] = u
            yt_ref[1] = s_a
            yc1_ref[0] = s_b
        else:
            yc2_ref[0] = u
            yc1_ref[1] = s_a
            yc2_ref[1] = s_b


def kernel(X_target, X_context1, X_context2, W1, b1, W2, b2):
    B, C, Hh, Ww = X_target.shape
    N = Hh * Ww
    xt = X_target.reshape(B, C, N)
    xc1 = X_context1.reshape(B, C, N)
    xc2 = X_context2.reshape(B, C, N)
    b1c = b1.reshape(C, 1)
    b2c = b2.reshape(C, 1)

    shp = jax.ShapeDtypeStruct((B, C, N), jnp.float32)
    yt, yc1, yc2 = pl.pallas_call(
        _hg_kernel,
        out_shape=[shp, shp, shp],
    )(xt, xc1, xc2, W1, b1c, W2, b2c)

    rs = lambda a: a.reshape(B, C, Hh, Ww)
    return (rs(yt), rs(yc1), rs(yc2))
